# baseline (speedup 1.0000x reference)
import jax
import jax.numpy as jnp
from jax.experimental import pallas as pl
from jax.experimental.pallas import tpu as pltpu


def _fakequant_block(scale_ref, zp_ref, x_ref, o_ref):
    s = scale_ref[0]
    zp = zp_ref[0]
    inv = 1.0 / s
    q = jnp.round(s * x_ref[...] - zp)
    o_ref[...] = (q + zp) * inv


def kernel(x, scale, zero_point):
    orig_shape = x.shape
    x2 = x.reshape(-1, x.shape[-1])
    R, C = x2.shape
    BR = 512
    out = pl.pallas_call(
        _fakequant_block,
        grid=(R // BR,),
        in_specs=[
            pl.BlockSpec(memory_space=pltpu.SMEM),
            pl.BlockSpec(memory_space=pltpu.SMEM),
            pl.BlockSpec((BR, C), lambda i: (i, 0)),
        ],
        out_specs=pl.BlockSpec((BR, C), lambda i: (i, 0)),
        out_shape=jax.ShapeDtypeStruct((R, C), x.dtype),
    )(scale, zero_point, x2)
    return out.reshape(orig_shape)
